# baseline (speedup 1.0000x reference)
import jax
import jax.numpy as jnp
from jax import lax
from jax.experimental import pallas as pl
from jax.experimental.pallas import tpu as pltpu
from jax.experimental.pallas import tpu_sc as plsc

_NC = 2
_NS = 16
_NW = _NC * _NS
_L = 16

_RPW = 320
_NPAD = _NW * _RPW
_ACC_R = _RPW + 16
_BLK = 2048
_C = 128
_SCAN = 4096


def _mesh():
    return plsc.VectorSubcoreMesh(core_axis_name="c", subcore_axis_name="s")


def _wid():
    return lax.axis_index("s") * _NC + lax.axis_index("c")


def _iota():
    return lax.iota(jnp.int32, _L)


def _make_binning(E_pad, CAP):
    NBLK = E_pad // _SCAN
    assert NBLK * _SCAN == E_pad and NBLK % 2 == 0

    out_type = [
        jax.ShapeDtypeStruct((_NW, CAP), jnp.int32),
        jax.ShapeDtypeStruct((_NW, CAP), jnp.int32),
        jax.ShapeDtypeStruct((_NW, 8), jnp.int32),
    ]
    scratch = [
        pltpu.VMEM((2, _SCAN), jnp.int32),
        pltpu.VMEM((2, _SCAN), jnp.int32),
        pltpu.VMEM((_BLK + 2 * _L,), jnp.int32),
        pltpu.VMEM((_BLK + 2 * _L,), jnp.int32),
        pltpu.VMEM((_L,), jnp.int32),
        pltpu.SemaphoreType.DMA,
        pltpu.SemaphoreType.DMA,
    ]

    def body(src_hbm, dst_hbm, sl_hbm, dl_hbm, cnt_hbm,
             sblk, dblk, sstage, dstage, cbuf, sema, semb):
        g = _wid()
        iota = _iota()

        def fire(b, slot, sem):
            e0 = pl.multiple_of(b * _SCAN, 8)
            pltpu.async_copy(src_hbm.at[pl.ds(e0, _SCAN)], sblk.at[slot], sem)
            pltpu.async_copy(dst_hbm.at[pl.ds(e0, _SCAN)], dblk.at[slot], sem)

        def wait(slot, sem):
            pltpu.make_async_copy(
                src_hbm.at[pl.ds(0, _SCAN)], sblk.at[slot], sem
            ).wait()
            pltpu.make_async_copy(
                dst_hbm.at[pl.ds(0, _SCAN)], dblk.at[slot], sem
            ).wait()

        def flush(fill, off):
            off = pl.multiple_of(off, 8)
            pltpu.sync_copy(
                sstage.at[pl.ds(0, _BLK)], sl_hbm.at[g, pl.ds(off, _BLK)]
            )
            pltpu.sync_copy(
                dstage.at[pl.ds(0, _BLK)], dl_hbm.at[g, pl.ds(off, _BLK)]
            )
            sstage[pl.ds(0, _L)] = sstage[pl.ds(_BLK, _L)]
            dstage[pl.ds(0, _L)] = dstage[pl.ds(_BLK, _L)]

        def scan_one(slot, carry):
            def group(gi, c):
                fill, off = c
                srcv = sblk[slot, pl.ds(gi * _L, _L)]
                dstv = dblk[slot, pl.ds(gi * _L, _L)]
                m = lax.div(dstv, jnp.full((_L,), _RPW, jnp.int32)) == g
                pos = fill + plsc.cumsum(m.astype(jnp.int32)) - 1
                plsc.store_scatter(sstage, [pos], srcv, mask=m)
                plsc.store_scatter(dstage, [pos], dstv, mask=m)
                fill = fill + plsc.all_reduce_population_count(m)[0]
                do = fill >= _BLK

                @pl.when(do)
                def _():
                    flush(fill, off)

                return (
                    jnp.where(do, fill - _BLK, fill),
                    jnp.where(do, off + _BLK, off),
                )

            return lax.fori_loop(0, _SCAN // _L, group, carry)

        fire(0, 0, sema)
        fire(1, 1, semb)

        def pair(p, carry):
            wait(0, sema)
            carry = scan_one(0, carry)

            @pl.when(p < NBLK // 2 - 1)
            def _():
                fire(2 * p + 2, 0, sema)

            wait(1, semb)
            carry = scan_one(1, carry)

            @pl.when(p < NBLK // 2 - 1)
            def _():
                fire(2 * p + 3, 1, semb)

            return carry

        fill, off = lax.fori_loop(0, NBLK // 2, pair, (0, 0))

        padv = g * _RPW + _RPW

        @pl.when(fill > 0)
        def _():
            def padgrp(j, _):
                v = sstage[pl.ds(j * _L, _L)]
                keep = (j * _L + iota) < fill
                sstage[pl.ds(j * _L, _L)] = jnp.where(keep, v, 0)
                w = dstage[pl.ds(j * _L, _L)]
                dstage[pl.ds(j * _L, _L)] = jnp.where(keep, w, padv)
                return 0

            lax.fori_loop(0, _BLK // _L, padgrp, 0)
            flush(fill, off)

        total = jnp.where(fill > 0, off + _BLK, off)
        cbuf[...] = jnp.where(iota == 0, total, 0)
        pltpu.sync_copy(cbuf.at[pl.ds(0, 8)], cnt_hbm.at[g])

    return pl.kernel(
        body,
        out_type=out_type,
        mesh=_mesh(),
        scratch_types=scratch,
        compiler_params=pltpu.CompilerParams(
            use_tc_tiling_on_sc=False, needs_layout_passes=False
        ),
    )


def _make_aggregate(H, CAP, with_deg):
    NCHB = _BLK // _C

    out_type = [jax.ShapeDtypeStruct((_NPAD, H), jnp.float32)]
    scratch = [
        pltpu.VMEM((NCHB, _C), jnp.int32),
        pltpu.VMEM((_BLK,), jnp.int32),
        pltpu.VMEM((2, _C, H), jnp.float32),
        pltpu.VMEM((_ACC_R, H), jnp.float32),
        pltpu.VMEM((_L,), jnp.int32),
        pltpu.SemaphoreType.DMA,
        pltpu.SemaphoreType.DMA,
        pltpu.SemaphoreType.DMA,
    ]
    if with_deg:
        out_type.append(jax.ShapeDtypeStruct((_NPAD,), jnp.float32))
        scratch.append(pltpu.VMEM((_ACC_R + _L,), jnp.float32))

    def body(*refs):
        if with_deg:
            (m_hbm, sl_hbm, dl_hbm, cnt_hbm, agg_hbm, deg_hbm,
             sidx, didx, rows, acc, csmem, sl_sem, ga, gb, dacc) = refs
        else:
            (m_hbm, sl_hbm, dl_hbm, cnt_hbm, agg_hbm,
             sidx, didx, rows, acc, csmem, sl_sem, ga, gb) = refs
        g = _wid()
        iota = _iota()
        zero = jnp.zeros((_L,), jnp.float32)
        onehot = jnp.where(iota == 0, jnp.float32(1.0), jnp.float32(0.0))

        pltpu.sync_copy(cnt_hbm.at[g], csmem.at[pl.ds(0, 8)])
        nblk = lax.div(csmem[...][0], _BLK)

        def z(i, _):
            for j in range(H // _L):
                acc[i, pl.ds(j * _L, _L)] = zero
            return 0

        lax.fori_loop(0, _ACC_R, z, 0)
        if with_deg:
            for j in range((_ACC_R + _L) // _L):
                dacc[pl.ds(j * _L, _L)] = zero

        base = g * _RPW

        def fire_rows(ch, buf, gsem):
            pltpu.async_copy(m_hbm.at[sidx.at[ch]], rows.at[buf], gsem)

        def wait_rows(buf, gsem):
            pltpu.make_async_copy(
                m_hbm.at[sidx.at[0]], rows.at[buf], gsem
            ).wait()

        def consume(ch, buf, gsem):
            wait_rows(buf, gsem)
            for grp in range(_C // _L):
                dv = didx[pl.ds(ch * _C + grp * _L, _L)] - base
                for e in range(_L):
                    rsp = lax.broadcast(dv[e], (_L,))
                    for j in range(H // _L):
                        plsc.addupdate_scatter(
                            acc,
                            [rsp, iota + j * _L],
                            rows[buf, grp * _L + e, pl.ds(j * _L, _L)],
                        )
                    if with_deg:
                        plsc.addupdate_scatter(dacc, [rsp + iota], onehot)

        def block(b, carry):
            o = pl.multiple_of(b * _BLK, 8)
            pltpu.sync_copy(
                sl_hbm.at[g, pl.ds(b * NCHB, NCHB)], sidx
            )
            pltpu.sync_copy(dl_hbm.at[g, pl.ds(o, _BLK)], didx)
            fire_rows(0, 0, ga)
            fire_rows(1, 1, gb)

            def cpair(q, c):
                ch = q * 2
                consume(ch, 0, ga)

                @pl.when(q + 1 < NCHB // 2)
                def _():
                    fire_rows(ch + 2, 0, ga)

                consume(ch + 1, 1, gb)

                @pl.when(q + 1 < NCHB // 2)
                def _():
                    fire_rows(ch + 3, 1, gb)

                return c

            return lax.fori_loop(0, NCHB // 2, cpair, carry)

        lax.fori_loop(0, nblk, block, 0)

        o0 = pl.multiple_of(base, 8)
        pltpu.sync_copy(acc.at[pl.ds(0, _RPW)], agg_hbm.at[pl.ds(o0, _RPW)])
        if with_deg:
            pltpu.sync_copy(
                dacc.at[pl.ds(0, _RPW)], deg_hbm.at[pl.ds(o0, _RPW)]
            )

    return pl.kernel(
        body,
        out_type=out_type,
        mesh=_mesh(),
        scratch_types=scratch,
        compiler_params=pltpu.CompilerParams(
            use_tc_tiling_on_sc=False, needs_layout_passes=False
        ),
    )


def _mm1_body(x_ref, w_ref, b_ref, o_ref):
    o_ref[...] = (
        jnp.dot(x_ref[...], w_ref[...], preferred_element_type=jnp.float32)
        + b_ref[...]
    )


def _norm_agg(p_ref, deg_ref):
    h = jnp.maximum(p_ref[...] / jnp.maximum(deg_ref[...], 1.0), 0.0)
    nrm = jnp.sqrt(jnp.sum(h * h, axis=1, keepdims=True))
    return h / (nrm + 1e-6)


def _layer_body(p_ref, deg_ref, w_ref, b_ref, o_ref):
    h = _norm_agg(p_ref, deg_ref)
    o_ref[...] = (
        jnp.dot(h, w_ref[...], preferred_element_type=jnp.float32) + b_ref[...]
    )


def _head_body(p_ref, deg_ref, w1_ref, b1_ref, w2_ref, b2_ref, o_ref):
    h = _norm_agg(p_ref, deg_ref)
    z = jnp.maximum(
        jnp.dot(h, w1_ref[...], preferred_element_type=jnp.float32) + b1_ref[...],
        0.0,
    )
    y = jnp.dot(z, w2_ref[...], preferred_element_type=jnp.float32) + b2_ref[...]
    o_ref[...] = 1.0 / (1.0 + jnp.exp(-y))


def _mm1(x, W, b, bm=2000):
    N, D = x.shape
    H = W.shape[1]
    return pl.pallas_call(
        _mm1_body,
        grid=(N // bm,),
        in_specs=[
            pl.BlockSpec((bm, D), lambda i: (i, 0)),
            pl.BlockSpec((D, H), lambda i: (0, 0)),
            pl.BlockSpec((1, H), lambda i: (0, 0)),
        ],
        out_specs=pl.BlockSpec((bm, H), lambda i: (i, 0)),
        out_shape=jax.ShapeDtypeStruct((N, H), jnp.float32),
    )(x, W, b.reshape(1, H))


def _layer(N, p, deg, W, b, bm=2000):
    H = p.shape[1]
    return pl.pallas_call(
        _layer_body,
        grid=(N // bm,),
        in_specs=[
            pl.BlockSpec((bm, H), lambda i: (i, 0)),
            pl.BlockSpec((bm, 1), lambda i: (i, 0)),
            pl.BlockSpec((H, H), lambda i: (0, 0)),
            pl.BlockSpec((1, H), lambda i: (0, 0)),
        ],
        out_specs=pl.BlockSpec((bm, H), lambda i: (i, 0)),
        out_shape=jax.ShapeDtypeStruct((N, H), jnp.float32),
    )(p, deg, W, b.reshape(1, H))


def _head(N, p, deg, Wm1, bm1, Wm2, bm2, bm=2000):
    H = p.shape[1]
    K = Wm1.shape[1]
    return pl.pallas_call(
        _head_body,
        grid=(N // bm,),
        in_specs=[
            pl.BlockSpec((bm, H), lambda i: (i, 0)),
            pl.BlockSpec((bm, 1), lambda i: (i, 0)),
            pl.BlockSpec((H, K), lambda i: (0, 0)),
            pl.BlockSpec((1, K), lambda i: (0, 0)),
            pl.BlockSpec((K, 1), lambda i: (0, 0)),
            pl.BlockSpec((1, 1), lambda i: (0, 0)),
        ],
        out_specs=pl.BlockSpec((bm, 1), lambda i: (i, 0)),
        out_shape=jax.ShapeDtypeStruct((N, 1), jnp.float32),
    )(p, deg, Wm1, bm1.reshape(1, K), Wm2, bm2.reshape(1, 1))


def kernel(x, edge_index, W1, b1, W2, b2, W3, b3, Wm1, bm1, Wm2, bm2):
    N, D = x.shape
    H = W1.shape[1]
    E = edge_index.shape[1]
    E_pad = -(-E // (2 * _SCAN)) * (2 * _SCAN)
    CAP = E_pad + _BLK
    src = jnp.concatenate([edge_index[0], jnp.zeros((E_pad - E,), jnp.int32)])
    dst = jnp.concatenate(
        [edge_index[1], jnp.full((E_pad - E,), _NPAD, jnp.int32)]
    )

    binning = _make_binning(E_pad, CAP)
    agg_first = _make_aggregate(H, CAP, with_deg=True)
    agg_rest = _make_aggregate(H, CAP, with_deg=False)

    slists, dlists, counts = binning(src, dst)
    slists = slists.reshape(_NW, CAP // _C, _C)
    m1 = _mm1(x, W1, b1)
    p1, deg = agg_first(m1, slists, dlists, counts)
    deg = deg.reshape(_NPAD, 1)
    m2 = _layer(N, p1, deg, W2, b2)
    (p2,) = agg_rest(m2, slists, dlists, counts)
    m3 = _layer(N, p2, deg, W3, b3)
    (p3,) = agg_rest(m3, slists, dlists, counts)
    return _head(N, p3, deg, Wm1, bm1, Wm2, bm2)


# speedup vs baseline: 1.0896x; 1.0896x over previous
import jax
import jax.numpy as jnp
from jax import lax
from jax.experimental import pallas as pl
from jax.experimental.pallas import tpu as pltpu
from jax.experimental.pallas import tpu_sc as plsc

_NC = 2
_NS = 16
_NW = _NC * _NS
_L = 16

_RPW = 320
_NPAD = _NW * _RPW
_ACC_R = _RPW + 16
_BLK = 2048
_C = 128
_SCAN = 4096


def _mesh():
    return plsc.VectorSubcoreMesh(core_axis_name="c", subcore_axis_name="s")


def _wid():
    return lax.axis_index("s") * _NC + lax.axis_index("c")


def _iota():
    return lax.iota(jnp.int32, _L)


def _make_binning(E_pad, CAP):
    NBLK = E_pad // _SCAN
    assert NBLK * _SCAN == E_pad and NBLK % 2 == 0

    out_type = [
        jax.ShapeDtypeStruct((_NW, CAP), jnp.int32),
        jax.ShapeDtypeStruct((_NW, CAP), jnp.int32),
        jax.ShapeDtypeStruct((_NW, 8), jnp.int32),
    ]
    scratch = [
        pltpu.VMEM((2, _SCAN), jnp.int32),
        pltpu.VMEM((2, _SCAN), jnp.int32),
        pltpu.VMEM((3 * _BLK + _L,), jnp.int32),
        pltpu.VMEM((3 * _BLK + _L,), jnp.int32),
        pltpu.VMEM((_L,), jnp.int32),
        pltpu.SemaphoreType.DMA,
        pltpu.SemaphoreType.DMA,
    ]

    def body(src_hbm, dst_hbm, sl_hbm, dl_hbm, cnt_hbm,
             sblk, dblk, sstage, dstage, cbuf, sema, semb):
        g = _wid()
        iota = _iota()

        def fire(b, slot, sem):
            e0 = pl.multiple_of(b * _SCAN, 8)
            pltpu.async_copy(src_hbm.at[pl.ds(e0, _SCAN)], sblk.at[slot], sem)
            pltpu.async_copy(dst_hbm.at[pl.ds(e0, _SCAN)], dblk.at[slot], sem)

        def wait(slot, sem):
            pltpu.make_async_copy(
                src_hbm.at[pl.ds(0, _SCAN)], sblk.at[slot], sem
            ).wait()
            pltpu.make_async_copy(
                dst_hbm.at[pl.ds(0, _SCAN)], dblk.at[slot], sem
            ).wait()

        def emit(s0, off):
            off = pl.multiple_of(off, 8)
            pltpu.sync_copy(
                sstage.at[pl.ds(s0, _BLK)], sl_hbm.at[g, pl.ds(off, _BLK)]
            )
            pltpu.sync_copy(
                dstage.at[pl.ds(s0, _BLK)], dl_hbm.at[g, pl.ds(off, _BLK)]
            )

        def drain(fillv, off):
            f = fillv[0]
            nf = lax.div(f, _BLK)

            @pl.when(nf >= 1)
            def _():
                emit(0, off)

            @pl.when(nf >= 2)
            def _():
                emit(_BLK, off + _BLK)

            @pl.when(nf > 0)
            def _():
                rbase = pl.multiple_of(nf * _BLK, 8)

                def mv(i, _):
                    sstage[pl.ds(i * _L, _L)] = sstage[pl.ds(rbase + i * _L, _L)]
                    dstage[pl.ds(i * _L, _L)] = dstage[pl.ds(rbase + i * _L, _L)]
                    return 0

                lax.fori_loop(0, _BLK // _L, mv, 0)

            return fillv - nf * _BLK, off + nf * _BLK

        def scan_one(slot, carry):
            fillv, off = carry

            def group(gi, fillv):
                srcv = sblk[slot, pl.ds(gi * _L, _L)]
                dstv = dblk[slot, pl.ds(gi * _L, _L)]
                m = lax.div(dstv, jnp.full((_L,), _RPW, jnp.int32)) == g
                pos = fillv + plsc.cumsum(m.astype(jnp.int32)) - 1
                plsc.store_scatter(sstage, [pos], srcv, mask=m)
                plsc.store_scatter(dstage, [pos], dstv, mask=m)
                return fillv + plsc.all_reduce_population_count(m)

            fillv = lax.fori_loop(0, _SCAN // _L, group, fillv)
            return drain(fillv, off)

        fire(0, 0, sema)
        fire(1, 1, semb)

        def pair(p, carry):
            wait(0, sema)
            carry = scan_one(0, carry)

            @pl.when(p < NBLK // 2 - 1)
            def _():
                fire(2 * p + 2, 0, sema)

            wait(1, semb)
            carry = scan_one(1, carry)

            @pl.when(p < NBLK // 2 - 1)
            def _():
                fire(2 * p + 3, 1, semb)

            return carry

        fillv, off = lax.fori_loop(
            0, NBLK // 2, pair, (jnp.zeros((_L,), jnp.int32), 0)
        )
        fill = fillv[0]

        padv = g * _RPW + _RPW

        @pl.when(fill > 0)
        def _():
            def padgrp(j, _):
                v = sstage[pl.ds(j * _L, _L)]
                keep = (j * _L + iota) < fillv
                sstage[pl.ds(j * _L, _L)] = jnp.where(keep, v, 0)
                w = dstage[pl.ds(j * _L, _L)]
                dstage[pl.ds(j * _L, _L)] = jnp.where(keep, w, padv)
                return 0

            lax.fori_loop(0, _BLK // _L, padgrp, 0)
            emit(0, off)

        total = jnp.where(fill > 0, off + _BLK, off)
        cbuf[...] = jnp.where(iota == 0, total, 0)
        pltpu.sync_copy(cbuf.at[pl.ds(0, 8)], cnt_hbm.at[g])

    return pl.kernel(
        body,
        out_type=out_type,
        mesh=_mesh(),
        scratch_types=scratch,
        compiler_params=pltpu.CompilerParams(
            use_tc_tiling_on_sc=False, needs_layout_passes=False
        ),
    )


def _make_aggregate(H, CAP, with_deg):
    NCHB = _BLK // _C

    out_type = [jax.ShapeDtypeStruct((_NPAD, H), jnp.float32)]
    scratch = [
        pltpu.VMEM((NCHB, _C), jnp.int32),
        pltpu.VMEM((_BLK,), jnp.int32),
        pltpu.VMEM((2, _C, H), jnp.float32),
        pltpu.VMEM((_ACC_R, H), jnp.float32),
        pltpu.VMEM((_L,), jnp.int32),
        pltpu.SemaphoreType.DMA,
        pltpu.SemaphoreType.DMA,
        pltpu.SemaphoreType.DMA,
    ]
    if with_deg:
        out_type.append(jax.ShapeDtypeStruct((_NPAD,), jnp.float32))
        scratch.append(pltpu.VMEM((_ACC_R + _L,), jnp.float32))

    def body(*refs):
        if with_deg:
            (m_hbm, sl_hbm, dl_hbm, cnt_hbm, agg_hbm, deg_hbm,
             sidx, didx, rows, acc, csmem, sl_sem, ga, gb, dacc) = refs
        else:
            (m_hbm, sl_hbm, dl_hbm, cnt_hbm, agg_hbm,
             sidx, didx, rows, acc, csmem, sl_sem, ga, gb) = refs
        g = _wid()
        iota = _iota()
        zero = jnp.zeros((_L,), jnp.float32)
        onehot = jnp.where(iota == 0, jnp.float32(1.0), jnp.float32(0.0))

        pltpu.sync_copy(cnt_hbm.at[g], csmem.at[pl.ds(0, 8)])
        nblk = lax.div(csmem[...][0], _BLK)

        def z(i, _):
            for j in range(H // _L):
                acc[i, pl.ds(j * _L, _L)] = zero
            return 0

        lax.fori_loop(0, _ACC_R, z, 0)
        if with_deg:
            for j in range((_ACC_R + _L) // _L):
                dacc[pl.ds(j * _L, _L)] = zero

        base = g * _RPW

        def fire_rows(ch, buf, gsem):
            pltpu.async_copy(m_hbm.at[sidx.at[ch]], rows.at[buf], gsem)

        def wait_rows(buf, gsem):
            pltpu.make_async_copy(
                m_hbm.at[sidx.at[0]], rows.at[buf], gsem
            ).wait()

        gdims = lax.GatherDimensionNumbers(
            offset_dims=(), collapsed_slice_dims=(0,), start_index_map=(0,)
        )

        def splat(v, e):
            return lax.gather(
                v,
                jnp.full((_L, 1), e, jnp.int32),
                gdims,
                (1,),
                mode=lax.GatherScatterMode.PROMISE_IN_BOUNDS,
            )

        def consume(ch, buf, gsem):
            wait_rows(buf, gsem)
            for grp in range(_C // _L):
                dv = didx[pl.ds(ch * _C + grp * _L, _L)] - base
                for e in range(_L):
                    rsp = splat(dv, e)
                    for j in range(H // _L):
                        plsc.addupdate_scatter(
                            acc,
                            [rsp, iota + j * _L],
                            rows[buf, grp * _L + e, pl.ds(j * _L, _L)],
                        )
                    if with_deg:
                        plsc.addupdate_scatter(dacc, [rsp + iota], onehot)

        def block(b, carry):
            o = pl.multiple_of(b * _BLK, 8)
            pltpu.sync_copy(
                sl_hbm.at[g, pl.ds(b * NCHB, NCHB)], sidx
            )
            pltpu.sync_copy(dl_hbm.at[g, pl.ds(o, _BLK)], didx)
            fire_rows(0, 0, ga)
            fire_rows(1, 1, gb)

            def cpair(q, c):
                ch = q * 2
                consume(ch, 0, ga)

                @pl.when(q + 1 < NCHB // 2)
                def _():
                    fire_rows(ch + 2, 0, ga)

                consume(ch + 1, 1, gb)

                @pl.when(q + 1 < NCHB // 2)
                def _():
                    fire_rows(ch + 3, 1, gb)

                return c

            return lax.fori_loop(0, NCHB // 2, cpair, carry)

        lax.fori_loop(0, nblk, block, 0)

        o0 = pl.multiple_of(base, 8)
        pltpu.sync_copy(acc.at[pl.ds(0, _RPW)], agg_hbm.at[pl.ds(o0, _RPW)])
        if with_deg:
            pltpu.sync_copy(
                dacc.at[pl.ds(0, _RPW)], deg_hbm.at[pl.ds(o0, _RPW)]
            )

    return pl.kernel(
        body,
        out_type=out_type,
        mesh=_mesh(),
        scratch_types=scratch,
        compiler_params=pltpu.CompilerParams(
            use_tc_tiling_on_sc=False, needs_layout_passes=False
        ),
    )


def _mm1_body(x_ref, w_ref, b_ref, o_ref):
    o_ref[...] = (
        jnp.dot(x_ref[...], w_ref[...], preferred_element_type=jnp.float32)
        + b_ref[...]
    )


def _norm_agg(p_ref, deg_ref):
    h = jnp.maximum(p_ref[...] / jnp.maximum(deg_ref[...], 1.0), 0.0)
    nrm = jnp.sqrt(jnp.sum(h * h, axis=1, keepdims=True))
    return h / (nrm + 1e-6)


def _layer_body(p_ref, deg_ref, w_ref, b_ref, o_ref):
    h = _norm_agg(p_ref, deg_ref)
    o_ref[...] = (
        jnp.dot(h, w_ref[...], preferred_element_type=jnp.float32) + b_ref[...]
    )


def _head_body(p_ref, deg_ref, w1_ref, b1_ref, w2_ref, b2_ref, o_ref):
    h = _norm_agg(p_ref, deg_ref)
    z = jnp.maximum(
        jnp.dot(h, w1_ref[...], preferred_element_type=jnp.float32) + b1_ref[...],
        0.0,
    )
    y = jnp.dot(z, w2_ref[...], preferred_element_type=jnp.float32) + b2_ref[...]
    o_ref[...] = 1.0 / (1.0 + jnp.exp(-y))


def _mm1(x, W, b, bm=2000):
    N, D = x.shape
    H = W.shape[1]
    return pl.pallas_call(
        _mm1_body,
        grid=(N // bm,),
        in_specs=[
            pl.BlockSpec((bm, D), lambda i: (i, 0)),
            pl.BlockSpec((D, H), lambda i: (0, 0)),
            pl.BlockSpec((1, H), lambda i: (0, 0)),
        ],
        out_specs=pl.BlockSpec((bm, H), lambda i: (i, 0)),
        out_shape=jax.ShapeDtypeStruct((N, H), jnp.float32),
    )(x, W, b.reshape(1, H))


def _layer(N, p, deg, W, b, bm=2000):
    H = p.shape[1]
    return pl.pallas_call(
        _layer_body,
        grid=(N // bm,),
        in_specs=[
            pl.BlockSpec((bm, H), lambda i: (i, 0)),
            pl.BlockSpec((bm, 1), lambda i: (i, 0)),
            pl.BlockSpec((H, H), lambda i: (0, 0)),
            pl.BlockSpec((1, H), lambda i: (0, 0)),
        ],
        out_specs=pl.BlockSpec((bm, H), lambda i: (i, 0)),
        out_shape=jax.ShapeDtypeStruct((N, H), jnp.float32),
    )(p, deg, W, b.reshape(1, H))


def _head(N, p, deg, Wm1, bm1, Wm2, bm2, bm=2000):
    H = p.shape[1]
    K = Wm1.shape[1]
    return pl.pallas_call(
        _head_body,
        grid=(N // bm,),
        in_specs=[
            pl.BlockSpec((bm, H), lambda i: (i, 0)),
            pl.BlockSpec((bm, 1), lambda i: (i, 0)),
            pl.BlockSpec((H, K), lambda i: (0, 0)),
            pl.BlockSpec((1, K), lambda i: (0, 0)),
            pl.BlockSpec((K, 1), lambda i: (0, 0)),
            pl.BlockSpec((1, 1), lambda i: (0, 0)),
        ],
        out_specs=pl.BlockSpec((bm, 1), lambda i: (i, 0)),
        out_shape=jax.ShapeDtypeStruct((N, 1), jnp.float32),
    )(p, deg, Wm1, bm1.reshape(1, K), Wm2, bm2.reshape(1, 1))


def kernel(x, edge_index, W1, b1, W2, b2, W3, b3, Wm1, bm1, Wm2, bm2):
    N, D = x.shape
    H = W1.shape[1]
    E = edge_index.shape[1]
    E_pad = -(-E // (2 * _SCAN)) * (2 * _SCAN)
    CAP = E_pad + _BLK
    src = jnp.concatenate([edge_index[0], jnp.zeros((E_pad - E,), jnp.int32)])
    dst = jnp.concatenate(
        [edge_index[1], jnp.full((E_pad - E,), _NPAD, jnp.int32)]
    )

    binning = _make_binning(E_pad, CAP)
    agg_first = _make_aggregate(H, CAP, with_deg=True)
    agg_rest = _make_aggregate(H, CAP, with_deg=False)

    slists, dlists, counts = binning(src, dst)
    slists = slists.reshape(_NW, CAP // _C, _C)
    m1 = _mm1(x, W1, b1)
    p1, deg = agg_first(m1, slists, dlists, counts)
    deg = deg.reshape(_NPAD, 1)
    m2 = _layer(N, p1, deg, W2, b2)
    (p2,) = agg_rest(m2, slists, dlists, counts)
    m3 = _layer(N, p2, deg, W3, b3)
    (p3,) = agg_rest(m3, slists, dlists, counts)
    return _head(N, p3, deg, Wm1, bm1, Wm2, bm2)


# speedup vs baseline: 4.3297x; 3.9737x over previous
import functools

import jax
import jax.numpy as jnp
from jax import lax
from jax.experimental import pallas as pl
from jax.experimental.pallas import tpu as pltpu
from jax.experimental.pallas import tpu_sc as plsc

_NC = 2
_NS = 16


_C = 128


def _make_aggregate(N, H, E_pad, with_deg):
    NW = _NC * _NS
    NCH = E_pad // (_C * NW)
    assert NCH * _C * NW == E_pad and NCH % 2 == 0
    assert NCH % 4 == 0
    NA = N + 8
    RP = (-(-NA // _NS) + 7) // 8 * 8
    Z_LAST = NA - (_NS - 1) * RP
    O_LAST = N - (_NS - 1) * RP
    assert Z_LAST > 0 and Z_LAST % 8 == 0 and O_LAST > 0 and O_LAST % 8 == 0

    mesh = plsc.VectorSubcoreMesh(core_axis_name="c", subcore_axis_name="s")

    out_type = [jax.ShapeDtypeStruct((_NC, N, H), jnp.float32)]
    scratch = [
        pltpu.VMEM((NCH, _C), jnp.int32),
        pltpu.VMEM((NCH, _C), jnp.int32),
        pltpu.VMEM((4, _C, H), jnp.float32),
        pltpu.VMEM_SHARED((NA, H), jnp.float32),
        pltpu.SemaphoreType.DMA,
        pltpu.SemaphoreType.DMA,
        pltpu.SemaphoreType.DMA,
        pltpu.SemaphoreType.DMA,
    ]
    if with_deg:
        out_type.append(jax.ShapeDtypeStruct((_NC, N), jnp.float32))
        scratch += [
            pltpu.VMEM((_C,), jnp.float32),
            pltpu.VMEM_SHARED((NA,), jnp.float32),
        ]

    def body(*refs):
        if with_deg:
            (m_hbm, src_hbm, dst_hbm, znh_hbm, zn_hbm,
             agg_hbm, deg_hbm, src2d, dst2d, rows, acc, s0, s1, s2, s3,
             ones_v, dacc) = refs
        else:
            (m_hbm, src_hbm, dst_hbm, znh_hbm,
             agg_hbm, src2d, dst2d, rows, acc, s0, s1, s2, s3) = refs
        sems = (s0, s1, s2, s3)
        cid = lax.axis_index("c")
        sid = lax.axis_index("s")
        wid = sid * _NC + cid

        c0 = pl.multiple_of(wid * NCH, 8)
        pltpu.sync_copy(src_hbm.at[pl.ds(c0, NCH)], src2d)
        pltpu.sync_copy(dst_hbm.at[pl.ds(c0, NCH)], dst2d)

        r0 = pl.multiple_of(sid * RP, 8)

        @pl.when(sid < _NS - 1)
        def _():
            pltpu.sync_copy(znh_hbm.at[pl.ds(r0, RP)], acc.at[pl.ds(r0, RP)])

        @pl.when(sid == _NS - 1)
        def _():
            t0 = (_NS - 1) * RP
            pltpu.sync_copy(
                znh_hbm.at[pl.ds(t0, Z_LAST)], acc.at[pl.ds(t0, Z_LAST)]
            )
        if with_deg:

            @pl.when(sid == 0)
            def _():
                pltpu.sync_copy(zn_hbm, dacc)

            for j in range(_C // 16):
                ones_v[pl.ds(j * 16, 16)] = jnp.ones((16,), jnp.float32)
        plsc.subcore_barrier()

        def gfire(i, k):
            pltpu.async_copy(m_hbm.at[src2d.at[i]], rows.at[k], sems[k])

        def gwait(k):
            pltpu.make_async_copy(
                m_hbm.at[src2d.at[0]], rows.at[k], sems[k]
            ).wait()

        def scat(i, k):
            pltpu.sync_copy(rows.at[k], acc.at[dst2d.at[i]], add=True)
            if with_deg:
                pltpu.sync_copy(ones_v, dacc.at[dst2d.at[i]], add=True)

        for k in range(4):
            gfire(k, k)

        def quad(q, carry):
            i = q * 4
            for k in range(4):
                gwait(k)
                scat(i + k, k)

                @pl.when(i + k + 4 < NCH)
                def _():
                    gfire(i + k + 4, k)

            return carry

        lax.fori_loop(0, NCH // 4, quad, 0)
        plsc.subcore_barrier()

        @pl.when(sid < _NS - 1)
        def _():
            pltpu.sync_copy(acc.at[pl.ds(r0, RP)], agg_hbm.at[cid, pl.ds(r0, RP)])

        @pl.when(sid == _NS - 1)
        def _():
            t0 = (_NS - 1) * RP
            pltpu.sync_copy(
                acc.at[pl.ds(t0, O_LAST)], agg_hbm.at[cid, pl.ds(t0, O_LAST)]
            )
        if with_deg:

            @pl.when(sid == 0)
            def _():
                pltpu.sync_copy(dacc.at[pl.ds(0, N)], deg_hbm.at[cid])

    return pl.kernel(
        body,
        out_type=out_type,
        mesh=mesh,
        scratch_types=scratch,
        compiler_params=pltpu.CompilerParams(use_tc_tiling_on_sc=False),
    )


def _mm1_body(x_ref, w_ref, b_ref, o_ref):
    o_ref[...] = (
        jnp.dot(x_ref[...], w_ref[...], preferred_element_type=jnp.float32)
        + b_ref[...]
    )


def _norm_from_partials(p_ref, deg_ref):
    agg = p_ref[0] + p_ref[1]
    deg = deg_ref[0] + deg_ref[1]
    h = jnp.maximum(agg / jnp.maximum(deg, 1.0), 0.0)
    nrm = jnp.sqrt(jnp.sum(h * h, axis=1, keepdims=True))
    return h / (nrm + 1e-6)


def _layer_body(p_ref, deg_ref, w_ref, b_ref, o_ref):
    h = _norm_from_partials(p_ref, deg_ref)
    o_ref[...] = (
        jnp.dot(h, w_ref[...], preferred_element_type=jnp.float32) + b_ref[...]
    )


def _head_body(p_ref, deg_ref, w1_ref, b1_ref, w2_ref, b2_ref, o_ref):
    h = _norm_from_partials(p_ref, deg_ref)
    z = jnp.maximum(
        jnp.dot(h, w1_ref[...], preferred_element_type=jnp.float32) + b1_ref[...],
        0.0,
    )
    y = jnp.dot(z, w2_ref[...], preferred_element_type=jnp.float32) + b2_ref[...]
    o_ref[...] = 1.0 / (1.0 + jnp.exp(-y))


def _mm1(x, W, b, bm=2000):
    N, D = x.shape
    H = W.shape[1]
    return pl.pallas_call(
        _mm1_body,
        grid=(N // bm,),
        in_specs=[
            pl.BlockSpec((bm, D), lambda i: (i, 0)),
            pl.BlockSpec((D, H), lambda i: (0, 0)),
            pl.BlockSpec((1, H), lambda i: (0, 0)),
        ],
        out_specs=pl.BlockSpec((bm, H), lambda i: (i, 0)),
        out_shape=jax.ShapeDtypeStruct((N, H), jnp.float32),
    )(x, W, b.reshape(1, H))


def _layer(p, deg, W, b, bm=2000):
    _, N, H = p.shape
    return pl.pallas_call(
        _layer_body,
        grid=(N // bm,),
        in_specs=[
            pl.BlockSpec((_NC, bm, H), lambda i: (0, i, 0)),
            pl.BlockSpec((_NC, bm, 1), lambda i: (0, i, 0)),
            pl.BlockSpec((H, H), lambda i: (0, 0)),
            pl.BlockSpec((1, H), lambda i: (0, 0)),
        ],
        out_specs=pl.BlockSpec((bm, H), lambda i: (i, 0)),
        out_shape=jax.ShapeDtypeStruct((N, H), jnp.float32),
    )(p, deg, W, b.reshape(1, H))


def _head(p, deg, Wm1, bm1, Wm2, bm2, bm=2000):
    _, N, H = p.shape
    K = Wm1.shape[1]
    return pl.pallas_call(
        _head_body,
        grid=(N // bm,),
        in_specs=[
            pl.BlockSpec((_NC, bm, H), lambda i: (0, i, 0)),
            pl.BlockSpec((_NC, bm, 1), lambda i: (0, i, 0)),
            pl.BlockSpec((H, K), lambda i: (0, 0)),
            pl.BlockSpec((1, K), lambda i: (0, 0)),
            pl.BlockSpec((K, 1), lambda i: (0, 0)),
            pl.BlockSpec((1, 1), lambda i: (0, 0)),
        ],
        out_specs=pl.BlockSpec((bm, 1), lambda i: (i, 0)),
        out_shape=jax.ShapeDtypeStruct((N, 1), jnp.float32),
    )(p, deg, Wm1, bm1.reshape(1, K), Wm2, bm2.reshape(1, 1))


def kernel(x, edge_index, W1, b1, W2, b2, W3, b3, Wm1, bm1, Wm2, bm2):
    N, D = x.shape
    H = W1.shape[1]
    E = edge_index.shape[1]
    NW = _NC * _NS
    ncw = -(-E // (_C * NW))
    ncw += ncw % 2
    E_pad = ncw * _C * NW
    src = jnp.concatenate(
        [edge_index[0], jnp.zeros((E_pad - E,), jnp.int32)]
    ).reshape(-1, _C)
    dst = jnp.concatenate(
        [edge_index[1], jnp.full((E_pad - E,), N, jnp.int32)]
    ).reshape(-1, _C)
    znh = jnp.zeros((N + 8, H), jnp.float32)
    zn = jnp.zeros((N + 8,), jnp.float32)

    agg_first = _make_aggregate(N, H, E_pad, with_deg=True)
    agg_rest = _make_aggregate(N, H, E_pad, with_deg=False)

    m1 = _mm1(x, W1, b1)
    p1, deg2 = agg_first(m1, src, dst, znh, zn)
    deg2 = deg2.reshape(_NC, N, 1)
    m2 = _layer(p1, deg2, W2, b2)
    (p2,) = agg_rest(m2, src, dst, znh)
    m3 = _layer(p2, deg2, W3, b3)
    (p3,) = agg_rest(m3, src, dst, znh)
    return _head(p3, deg2, Wm1, bm1, Wm2, bm2)
